# baseline (speedup 1.0000x reference)
import jax
import jax.numpy as jnp
from jax import lax
from jax.experimental import pallas as pl
from jax.experimental.pallas import tpu as pltpu

N = 100000
DIM_H = 512
NUM_GRAPHS = 256
ROWS = 5000
NB = N // ROWS


def _fused_body(batch_ref, h_ref, W1_ref, b1_ref, W2_ref, b2_ref,
                out_ref, acc_ref, vn_ref):
    i = pl.program_id(0)

    @pl.when(i == 0)
    def _init():
        acc_ref[...] = jnp.zeros_like(acc_ref)

    @pl.when(i < NB)
    def _pool():
        ids = batch_ref[0, 0, :]
        seg = lax.broadcasted_iota(jnp.int32, (NUM_GRAPHS, ROWS), 0)
        onehot = (ids[None, :] == seg).astype(jnp.bfloat16)
        acc_ref[...] += jnp.dot(onehot, h_ref[...].astype(jnp.bfloat16),
                                preferred_element_type=jnp.float32)

    @pl.when(i == NB - 1)
    def _ffn():
        s = acc_ref[...]
        z = jnp.maximum(jnp.dot(s, W1_ref[...],
                                preferred_element_type=jnp.float32)
                        + b1_ref[...], 0.0)
        vn_ref[...] = jnp.dot(z, W2_ref[...],
                              preferred_element_type=jnp.float32) + b2_ref[...]

    @pl.when(i >= NB)
    def _broadcast():
        ids = batch_ref[0, 0, :]
        seg = lax.broadcasted_iota(jnp.int32, (ROWS, NUM_GRAPHS), 1)
        onehot = (ids[:, None] == seg).astype(jnp.bfloat16)
        out_ref[...] = jnp.dot(onehot, vn_ref[...].astype(jnp.bfloat16),
                               preferred_element_type=jnp.float32)


@jax.jit
def kernel(h, batch, W1, b1, W2, b2):
    batch3 = batch.astype(jnp.int32).reshape(NB, 1, ROWS)

    out = pl.pallas_call(
        _fused_body,
        grid=(2 * NB,),
        in_specs=[
            pl.BlockSpec((1, 1, ROWS),
                         lambda i: (jnp.where(i < NB, i, i - NB), 0, 0)),
            pl.BlockSpec((ROWS, DIM_H),
                         lambda i: (jnp.minimum(i, NB - 1), 0)),
            pl.BlockSpec((DIM_H, 2 * DIM_H), lambda i: (0, 0)),
            pl.BlockSpec((2 * DIM_H,), lambda i: (0,)),
            pl.BlockSpec((2 * DIM_H, DIM_H), lambda i: (0, 0)),
            pl.BlockSpec((DIM_H,), lambda i: (0,)),
        ],
        out_specs=pl.BlockSpec((ROWS, DIM_H),
                               lambda i: (jnp.maximum(i - NB, 0), 0)),
        out_shape=jax.ShapeDtypeStruct((N, DIM_H), jnp.float32),
        scratch_shapes=[pltpu.VMEM((NUM_GRAPHS, DIM_H), jnp.float32),
                        pltpu.VMEM((NUM_GRAPHS, DIM_H), jnp.float32)],
    )(batch3, h, W1, b1, W2, b2)
    return out
